# baseline (speedup 1.0000x reference)
import functools

import jax
import jax.numpy as jnp
from jax import lax
from jax.experimental import pallas as pl
from jax.experimental.pallas import tpu as pltpu
from jax.experimental.pallas import tpu_sc as plsc

SEQ = 2048
HIDDEN = 64
VOCAB = 512
B = SEQ * SEQ
ROW = 128
NROWS = B // ROW
NW = 16
ROWS_PER_W = NROWS // NW
K = 4
NG = ROWS_PER_W // K
NBUF = 2


def _make_gather():
    mesh = plsc.VectorSubcoreMesh(core_axis_name="c", subcore_axis_name="s")

    @functools.partial(
        pl.kernel,
        mesh=mesh,
        out_type=jax.ShapeDtypeStruct((B, HIDDEN), jnp.float32),
        scratch_types=[
            pltpu.VMEM((K, ROW), jnp.int32),
            pltpu.VMEM((K, ROW), jnp.int32),
            pltpu.VMEM((K * ROW, HIDDEN), jnp.float32),
            pltpu.VMEM((K * ROW, HIDDEN), jnp.float32),
            pltpu.VMEM_SHARED((VOCAB, HIDDEN), jnp.float32),
            pltpu.SemaphoreType.DMA,
            pltpu.SemaphoreType.DMA,
            pltpu.SemaphoreType.DMA,
            pltpu.SemaphoreType.DMA,
            pltpu.SemaphoreType.DMA,
            pltpu.SemaphoreType.DMA,
        ],
        compiler_params=pltpu.CompilerParams(
            use_tc_tiling_on_sc=False, disable_bounds_checks=True),
    )
    def gather_kernel(table_hbm, idx_hbm, out_hbm,
                      idx_v0, idx_v1, rows_v0, rows_v1, table_sp,
                      si0, si1, sg0, sg1, so0, so1):
        idx_bufs = (idx_v0, idx_v1)
        rows_bufs = (rows_v0, rows_v1)
        sem_i = (si0, si1)
        sem_g = (sg0, sg1)
        sem_o = (so0, so1)

        c = lax.axis_index("c")
        s = lax.axis_index("s")
        wid = s * 2 + c
        base_row = wid * ROWS_PER_W

        @pl.when(s == 0)
        def _():
            pltpu.sync_copy(table_hbm, table_sp)

        plsc.subcore_barrier()

        active = s < 8

        def idx_start(g, p):
            pltpu.async_copy(
                idx_hbm.at[pl.ds(base_row + g * K, K)], idx_bufs[p], sem_i[p])

        def idx_wait(p):
            pltpu.make_async_copy(
                idx_hbm.at[pl.ds(0, K)], idx_bufs[p], sem_i[p]).wait()

        def gather_start(p):
            for j in range(K):
                pltpu.async_copy(
                    table_sp.at[idx_bufs[p].at[j]],
                    rows_bufs[p].at[pl.ds(j * ROW, ROW)],
                    sem_g[p])

        def gather_wait(p):
            for j in range(K):
                pltpu.make_async_copy(
                    table_sp.at[idx_bufs[p].at[j]],
                    rows_bufs[p].at[pl.ds(j * ROW, ROW)],
                    sem_g[p]).wait()

        def out_start(g, p):
            pltpu.async_copy(
                rows_bufs[p],
                out_hbm.at[pl.ds((base_row + g * K) * ROW, K * ROW)],
                sem_o[p])

        def out_wait(g, p):
            pltpu.make_async_copy(
                rows_bufs[p],
                out_hbm.at[pl.ds((base_row + g * K) * ROW, K * ROW)],
                sem_o[p]).wait()

        @pl.when(active)
        def _():
            idx_start(0, 0)
            idx_start(1, 1)
            idx_wait(0)
            gather_start(0)

        def group(g, p):
            @pl.when(g + 1 < NG)
            def _():
                idx_wait(1 - p)

                @pl.when(g >= 1)
                def _():
                    out_wait(g - 1, 1 - p)

                gather_start(1 - p)

            gather_wait(p)
            out_start(g, p)

            @pl.when(g + 2 < NG)
            def _():
                idx_start(g + 2, p)

        def outer(gg, carry):
            for p in range(NBUF):
                group(gg * NBUF + p, p)
            return carry

        @pl.when(active)
        def _():
            lax.fori_loop(0, NG // NBUF, outer, 0)

            out_wait(NG - 2, 0)
            out_wait(NG - 1, 1)

    return gather_kernel


_gather = _make_gather()


def kernel(dist_mat, table):
    idx = dist_mat.astype(jnp.int32).reshape(NROWS, ROW)
    out = _gather(table, idx)
    return out.reshape(SEQ, SEQ, HIDDEN)


# speedup vs baseline: 1.0878x; 1.0878x over previous
import functools

import jax
import jax.numpy as jnp
from jax import lax
from jax.experimental import pallas as pl
from jax.experimental.pallas import tpu as pltpu
from jax.experimental.pallas import tpu_sc as plsc

SEQ = 2048
HIDDEN = 64
VOCAB = 512
B = SEQ * SEQ
ROW = 128
NROWS = B // ROW
NW = 32
ROWS_PER_W = NROWS // NW
K = 4
NG = ROWS_PER_W // K
NBUF = 2


def _make_gather():
    mesh = plsc.VectorSubcoreMesh(core_axis_name="c", subcore_axis_name="s")

    @functools.partial(
        pl.kernel,
        mesh=mesh,
        out_type=jax.ShapeDtypeStruct((B, HIDDEN), jnp.float32),
        scratch_types=[
            pltpu.VMEM((K, ROW), jnp.int32),
            pltpu.VMEM((K, ROW), jnp.int32),
            pltpu.VMEM((K * ROW, HIDDEN), jnp.float32),
            pltpu.VMEM((K * ROW, HIDDEN), jnp.float32),
            pltpu.VMEM_SHARED((VOCAB, HIDDEN), jnp.float32),
            pltpu.SemaphoreType.DMA,
            pltpu.SemaphoreType.DMA,
            pltpu.SemaphoreType.DMA,
            pltpu.SemaphoreType.DMA,
            pltpu.SemaphoreType.DMA,
            pltpu.SemaphoreType.DMA,
        ],
        compiler_params=pltpu.CompilerParams(
            use_tc_tiling_on_sc=False, disable_bounds_checks=True),
    )
    def gather_kernel(table_hbm, idx_hbm, out_hbm,
                      idx_v0, idx_v1, rows_v0, rows_v1, table_sp,
                      si0, si1, sg0, sg1, so0, so1):
        idx_bufs = (idx_v0, idx_v1)
        rows_bufs = (rows_v0, rows_v1)
        sem_i = (si0, si1)
        sem_g = (sg0, sg1)
        sem_o = (so0, so1)

        c = lax.axis_index("c")
        s = lax.axis_index("s")
        wid = s * 2 + c
        base_row = wid * ROWS_PER_W

        @pl.when(s == 0)
        def _():
            pltpu.sync_copy(table_hbm, table_sp)

        plsc.subcore_barrier()

        def idx_start(g, p):
            pltpu.async_copy(
                idx_hbm.at[pl.ds(base_row + g * K, K)], idx_bufs[p], sem_i[p])

        def idx_wait(p):
            pltpu.make_async_copy(
                idx_hbm.at[pl.ds(0, K)], idx_bufs[p], sem_i[p]).wait()

        def gather_start(p):
            for j in range(K):
                pltpu.async_copy(
                    table_sp.at[idx_bufs[p].at[j]],
                    rows_bufs[p].at[pl.ds(j * ROW, ROW)],
                    sem_g[p])

        def gather_wait(p):
            for j in range(K):
                pltpu.make_async_copy(
                    table_sp.at[idx_bufs[p].at[j]],
                    rows_bufs[p].at[pl.ds(j * ROW, ROW)],
                    sem_g[p]).wait()

        def out_start(g, p):
            pltpu.async_copy(
                rows_bufs[p],
                out_hbm.at[pl.ds((base_row + g * K) * ROW, K * ROW)],
                sem_o[p])

        def out_wait(g, p):
            pltpu.make_async_copy(
                rows_bufs[p],
                out_hbm.at[pl.ds((base_row + g * K) * ROW, K * ROW)],
                sem_o[p]).wait()

        idx_start(0, 0)
        idx_start(1, 1)
        idx_wait(0)
        gather_start(0)

        def group(g, p):
            @pl.when(g + 1 < NG)
            def _():
                idx_wait(1 - p)

                @pl.when(g >= 1)
                def _():
                    out_wait(g - 1, 1 - p)

                gather_start(1 - p)

            gather_wait(p)
            out_start(g, p)

            @pl.when(g + 2 < NG)
            def _():
                idx_start(g + 2, p)

        def outer(gg, carry):
            for p in range(NBUF):
                group(gg * NBUF + p, p)
            return carry

        lax.fori_loop(0, NG // NBUF, outer, 0)

        out_wait(NG - 2, 0)
        out_wait(NG - 1, 1)

    return gather_kernel


_gather = _make_gather()


def kernel(dist_mat, table):
    idx = dist_mat.astype(jnp.int32).reshape(NROWS, ROW)
    out = _gather(table, idx)
    return out.reshape(SEQ, SEQ, HIDDEN)
